# Initial kernel scaffold; baseline (speedup 1.0000x reference)
import jax
import jax.numpy as jnp
from jax import lax
from jax.experimental import pallas as pl
from jax.experimental.pallas import tpu as pltpu
from jax.experimental.pallas import tpu_sc as plsc

B = 4096
S = 200
D = 64
N = B * S
NC = 2
NS = 16
NW = NC * NS
ROWS_PER_W = N // NW
CHUNK = 800
NCHUNK = ROWS_PER_W // CHUNK
LANES = 16
DG = D // LANES


def _embed_body(idx_hbm, table_hbm, pos_hbm, out_hbm, idx_v, rows_v, pos_v, sem):
    wid = lax.axis_index("s") * NC + lax.axis_index("c")
    base = wid * ROWS_PER_W
    pltpu.sync_copy(pos_hbm, pos_v)

    def chunk_body(g, carry):
        off = base + g * CHUNK
        pltpu.sync_copy(idx_hbm.at[pl.ds(off, CHUNK)], idx_v)
        pltpu.async_copy(table_hbm.at[idx_v], rows_v, sem).wait()

        def s_body(s, c):
            for d in range(DG):
                pv = pos_v[s, pl.ds(d * LANES, LANES)]
                for q in range(CHUNK // S):
                    plsc.addupdate(rows_v.at[q * S + s, pl.ds(d * LANES, LANES)], pv)
            return c

        lax.fori_loop(0, S, s_body, 0)
        pltpu.sync_copy(rows_v, out_hbm.at[pl.ds(off, CHUNK)])
        return carry

    lax.fori_loop(0, NCHUNK, chunk_body, 0)


def kernel(batch_seqs, item_emb, pos_weight):
    idx = batch_seqs.reshape(N)
    k = pl.kernel(
        _embed_body,
        out_type=jax.ShapeDtypeStruct((N, D), jnp.float32),
        mesh=plsc.VectorSubcoreMesh(core_axis_name="c", subcore_axis_name="s"),
        scratch_types=[
            pltpu.VMEM((CHUNK,), jnp.int32),
            pltpu.VMEM((CHUNK, D), jnp.float32),
            pltpu.VMEM((S, D), jnp.float32),
            pltpu.SemaphoreType.DMA,
        ],
    )
    out = k(idx, item_emb, pos_weight)
    return out.reshape(B, S, D)


# speedup vs baseline: 3.7117x; 3.7117x over previous
import jax
import jax.numpy as jnp
from jax import lax
from jax.experimental import pallas as pl
from jax.experimental.pallas import tpu as pltpu
from jax.experimental.pallas import tpu_sc as plsc

B = 4096
S = 200
D = 64
N = B * S
NC = 2
NS = 16
NW = NC * NS
ROWS_PER_W = N // NW
CHUNK = 800
NCHUNK = ROWS_PER_W // CHUNK
LANES = 16
DG = D // LANES


def _embed_body(idx_hbm, table_hbm, pos_hbm, out_hbm, idx_v, rows_v, pos_v, sem):
    wid = lax.axis_index("s") * NC + lax.axis_index("c")
    base = wid * ROWS_PER_W
    pltpu.sync_copy(pos_hbm, pos_v)

    def chunk_body(g, carry):
        off = base + g * CHUNK
        pltpu.sync_copy(idx_hbm.at[pl.ds(off, CHUNK)], idx_v)
        pltpu.async_copy(table_hbm.at[idx_v], rows_v, sem).wait()

        def s_body(s, c):
            for d in range(DG):
                pv = pos_v[s, pl.ds(d * LANES, LANES)]
                for q in range(CHUNK // S):
                    plsc.addupdate(rows_v.at[q * S + s, pl.ds(d * LANES, LANES)], pv)
            return c

        lax.fori_loop(0, S, s_body, 0)
        pltpu.sync_copy(rows_v, out_hbm.at[pl.ds(off, CHUNK)])
        return carry

    lax.fori_loop(0, NCHUNK, chunk_body, 0)


def kernel(batch_seqs, item_emb, pos_weight):
    idx = batch_seqs.reshape(N)
    k = pl.kernel(
        _embed_body,
        out_type=jax.ShapeDtypeStruct((N, D), jnp.float32),
        mesh=plsc.VectorSubcoreMesh(core_axis_name="c", subcore_axis_name="s"),
        compiler_params=pltpu.CompilerParams(use_tc_tiling_on_sc=False),
        scratch_types=[
            pltpu.VMEM((CHUNK,), jnp.int32),
            pltpu.VMEM((CHUNK, D), jnp.float32),
            pltpu.VMEM((S, D), jnp.float32),
            pltpu.SemaphoreType.DMA,
        ],
    )
    out = k(idx, item_emb, pos_weight)
    return out.reshape(B, S, D)
